# Initial kernel scaffold; baseline (speedup 1.0000x reference)
import jax
import jax.numpy as jnp
from jax.experimental import pallas as pl


def kernel(inputs, emb_tables, W1, b1, W2, b2, Wo, bo):
    raise NotImplementedError("write your pallas kernel here")


# speedup vs baseline: 1.8901x; 1.8901x over previous
import functools

import jax
import jax.numpy as jnp
from jax import lax
from jax.experimental import pallas as pl
from jax.experimental.pallas import tpu as pltpu
from jax.experimental.pallas import tpu_sc as plsc

B = 4096
N_DENSE = 13
N_SPARSE = 26
VOCAB = 100000
D = 16

NUM_CORES = 2
NUM_SUBCORES = 16
NUM_WORKERS = NUM_CORES * NUM_SUBCORES
ROWS_PER_W = B // NUM_WORKERS
SLAB = ROWS_PER_W * N_SPARSE
GATHER_CHUNK = 128
N_CHUNKS = SLAB // GATHER_CHUNK


def _sc_att(sparse_flat, offs, table):
    mesh = plsc.VectorSubcoreMesh(core_axis_name="c", subcore_axis_name="s")

    @functools.partial(
        pl.kernel,
        mesh=mesh,
        out_type=jax.ShapeDtypeStruct((B, D), jnp.float32),
        compiler_params=pltpu.CompilerParams(use_tc_tiling_on_sc=False),
        scratch_types=[
            pltpu.VMEM((SLAB,), jnp.int32),
            pltpu.VMEM((SLAB,), jnp.int32),
            pltpu.VMEM((SLAB, D), jnp.float32),
            pltpu.VMEM((ROWS_PER_W, D), jnp.float32),
            pltpu.SemaphoreType.DMA,
        ],
    )
    def body(sparse_hbm, offs_hbm, table_hbm, att_hbm,
             idx_v, offs_v, rows_v, att_v, sem):
        cid = lax.axis_index("c")
        sid = lax.axis_index("s")
        wid = sid * NUM_CORES + cid
        base = wid * SLAB

        pltpu.sync_copy(sparse_hbm.at[pl.ds(base, SLAB)], idx_v)
        pltpu.sync_copy(offs_hbm, offs_v)

        def add_offs(i, carry):
            sl = pl.ds(i * 16, 16)
            idx_v[sl] = idx_v[sl] + offs_v[sl]
            return carry
        lax.fori_loop(0, SLAB // 16, add_offs, 0)

        copies = []
        for k in range(N_CHUNKS):
            sl = pl.ds(k * GATHER_CHUNK, GATHER_CHUNK)
            cp = pltpu.make_async_copy(table_hbm.at[idx_v.at[sl]],
                                       rows_v.at[sl], sem)
            cp.start()
            copies.append(cp)
        for cp in copies:
            cp.wait()

        def row_body(r, carry):
            j = r * N_SPARSE
            v = rows_v[j]
            s_acc = v
            q_acc = v * v
            for f in range(1, N_SPARSE):
                v = rows_v[j + f]
                s_acc = s_acc + v
                q_acc = q_acc + v * v
            att_v[r] = 0.5 * (s_acc * s_acc - q_acc)
            return carry
        lax.fori_loop(0, ROWS_PER_W, row_body, 0)

        pltpu.sync_copy(att_v, att_hbm.at[pl.ds(wid * ROWS_PER_W, ROWS_PER_W)])

    return body(sparse_flat, offs, table)


def _tc_head(att, wo_row, bo):
    def body(att_ref, wo_ref, bo_ref, out_ref):
        att_b = att_ref[...]
        wo = wo_ref[...]
        logit = jnp.sum(att_b * wo, axis=1, keepdims=True) + bo_ref[...]
        out_ref[...] = jax.nn.sigmoid(logit)

    return pl.pallas_call(
        body,
        out_shape=jax.ShapeDtypeStruct((B, 1), jnp.float32),
    )(att, wo_row, bo)


def kernel(inputs, emb_tables, W1, b1, W2, b2, Wo, bo):
    del W1, b1, W2, b2
    sparse_flat = inputs[:, N_DENSE:].reshape(-1)
    offs = (jnp.arange(SLAB, dtype=jnp.int32) % N_SPARSE) * VOCAB
    table = emb_tables.reshape(N_SPARSE * VOCAB, D)
    att = _sc_att(sparse_flat, offs, table)
    return _tc_head(att, Wo.reshape(1, D), bo.reshape(1, 1))
